# baseline (speedup 1.0000x reference)
import functools

import jax
import jax.numpy as jnp
from jax import lax
from jax.experimental import pallas as pl
from jax.experimental.pallas import tpu as pltpu
from jax.experimental.pallas import tpu_sc as plsc

_B, _L, _V, _D, _C = 4096, 50, 1000000, 64, 5
_N = _B * _L
_NW = 32
_ROWS_PER_W = _N // _NW
_CHUNK = 128
_CHUNKS = _ROWS_PER_W // _CHUNK
_NBUF = 5

_PC = 32
_VB = 256
_PGRID = 977
_P = _PGRID * _VB
_TBLOCKS = (_V + _VB - 1) // _VB


def _proj_body(t0_ref, t1_ref, t2_ref, t3_ref, w_ref, e_ref, out_ref):
    pieces = []
    for t_ref in (t0_ref, t1_ref, t2_ref, t3_ref):
        pj = jnp.dot(w_ref[...], t_ref[...], preferred_element_type=jnp.float32)
        pieces.append(
            lax.dot_general(pj, e_ref[...], (((0,), (0,)), ((), ())),
                            preferred_element_type=jnp.float32)
        )
    out_ref[...] = jnp.concatenate(pieces, axis=1)


def _project_table(table_t, w32t, eye32):
    def in_spec(j):
        return pl.BlockSpec(
            (_D, _VB),
            lambda g, j=j: (0, jnp.minimum(g + j * _PGRID, _TBLOCKS - 1)),
        )

    return pl.pallas_call(
        _proj_body,
        grid=(_PGRID,),
        in_specs=[
            in_spec(0), in_spec(1), in_spec(2), in_spec(3),
            pl.BlockSpec((_PC, _D), lambda g: (0, 0)),
            pl.BlockSpec((_PC, _PC), lambda g: (0, 0)),
        ],
        out_specs=pl.BlockSpec((_VB, 4 * _PC), lambda g: (g, 0)),
        out_shape=jax.ShapeDtypeStruct((_P, 4 * _PC), jnp.float32),
    )(table_t, table_t, table_t, table_t, w32t, eye32)


def _gather_body(idx_hbm, ptab_hbm, out_hbm, idx_v, buf_v, gsem):
    cid = lax.axis_index("c")
    sid = lax.axis_index("s")
    wid = sid * 2 + cid
    base = wid * _ROWS_PER_W
    pltpu.sync_copy(idx_hbm.at[wid], idx_v)

    def fire(j, slot):
        pltpu.async_copy(ptab_hbm.at[idx_v.at[j]], buf_v.at[slot], gsem.at[slot])

    for s in range(_NBUF):
        fire(s, s)

    def outer(j0, carry):
        for s in range(_NBUF):
            j = j0 * _NBUF + s
            pltpu.make_async_copy(
                ptab_hbm.at[idx_v.at[j]], buf_v.at[s], gsem.at[s]
            ).wait()
            pltpu.sync_copy(buf_v.at[s], out_hbm.at[pl.ds(base + j * _CHUNK, _CHUNK)])

            @pl.when(j + _NBUF < _CHUNKS)
            def _():
                fire(j + _NBUF, s)

        return carry

    lax.fori_loop(0, _CHUNKS // _NBUF, outer, 0)


@functools.cache
def _sc_gather_fn():
    return pl.kernel(
        _gather_body,
        out_type=jax.ShapeDtypeStruct((_N, _PC), jnp.float32),
        mesh=plsc.VectorSubcoreMesh(core_axis_name="c", subcore_axis_name="s"),
        scratch_types=[
            pltpu.VMEM((_CHUNKS, _CHUNK), jnp.int32),
            pltpu.VMEM((_NBUF, _CHUNK, _PC), jnp.float32),
            pltpu.SemaphoreType.DMA((_NBUF,)),
        ],
        compiler_params=pltpu.CompilerParams(use_tc_tiling_on_sc=False),
    )


_BQ = 256


def _win_body(x_ref, b_ref, o0_ref, o1_ref, o2_ref, o3_ref):
    xb = x_ref[...]
    z2 = jnp.zeros((2, _BQ, 4 * _PC), jnp.float32)
    ppad = jnp.concatenate([z2, xb[: _L - 2], z2, xb[_L - 2 :]], axis=0)
    outs = (o0_ref, o1_ref, o2_ref, o3_ref)
    for j in range(4):
        acc = ppad[0:_L, :, 32 * j : 32 * j + _C]
        for i in range(1, 5):
            o = 32 * j + 6 * i
            acc = acc + ppad[i : i + _L, :, o : o + _C]
        acc = acc + b_ref[...]
        outs[j][...] = jnp.transpose(acc, (2, 0, 1))


def _window_sum(x4, bias):
    grid = _B // 4 // _BQ
    ospec = pl.BlockSpec((_C, _L, _BQ), lambda g: (0, 0, g))
    oshape = jax.ShapeDtypeStruct((_C, _L, _B // 4), jnp.float32)
    return pl.pallas_call(
        _win_body,
        grid=(grid,),
        in_specs=[
            pl.BlockSpec((_L, _BQ, 4 * _PC), lambda g: (0, g, 0)),
            pl.BlockSpec((1, 1, _C), lambda g: (0, 0, 0)),
        ],
        out_specs=[ospec, ospec, ospec, ospec],
        out_shape=[oshape, oshape, oshape, oshape],
    )(x4, bias)


def kernel(word, embed_table, W, b):
    table_t = jnp.transpose(embed_table)
    w_taps = W.reshape(_C, 5, _D)
    w32t = jnp.zeros((_PC, _D), jnp.float32)
    for i in range(5):
        w32t = w32t.at[6 * i : 6 * i + _C, :].set(w_taps[:, i, :])
    eye32 = jnp.eye(_PC, dtype=jnp.float32)

    ptab = _project_table(table_t, w32t, eye32)
    ptab_lin = ptab.reshape(4 * _P, _PC)

    word_t = jnp.transpose(word)
    wq = jnp.transpose(word_t.reshape(_L, 4, _B // 4), (0, 2, 1))
    v = wq.astype(jnp.int32).reshape(-1)
    idx = 4 * (v % _P) + v // _P
    idx = idx.reshape(_NW, _CHUNKS, _CHUNK)

    x = _sc_gather_fn()(idx, ptab_lin)
    x4 = x.reshape(_L, _B // 4, 4 * _PC)

    outs = _window_sum(x4, b.reshape(1, 1, _C))
    out_t = jnp.concatenate(outs, axis=2)
    return jnp.transpose(out_t, (2, 1, 0))


# speedup vs baseline: 2.4183x; 2.4183x over previous
import functools

import jax
import jax.numpy as jnp
from jax import lax
from jax.experimental import pallas as pl
from jax.experimental.pallas import tpu as pltpu
from jax.experimental.pallas import tpu_sc as plsc

_B, _L, _V, _D, _C = 4096, 50, 1000000, 64, 5
_N = _B * _L
_NW = 32
_ROWS_PER_W = _N // _NW
_CHUNK = 128
_CHUNKS = _ROWS_PER_W // _CHUNK
_NBUF = 5

_PC = 32
_VB = 1024
_PGRID = 245
_P = _PGRID * _VB
_TBLOCKS = (_V + _VB - 1) // _VB


def _proj_body(t0_ref, t1_ref, t2_ref, t3_ref, w_ref, out_ref):
    pieces = []
    for t_ref in (t0_ref, t1_ref, t2_ref, t3_ref):
        pieces.append(
            lax.dot_general(t_ref[...], w_ref[...], (((0,), (1,)), ((), ())),
                            preferred_element_type=jnp.float32)
        )
    out_ref[...] = jnp.concatenate(pieces, axis=1)


def _project_table(table_t, w32t):
    def in_spec(j):
        return pl.BlockSpec(
            (_D, _VB),
            lambda g, j=j: (0, jnp.minimum(g + j * _PGRID, _TBLOCKS - 1)),
        )

    return pl.pallas_call(
        _proj_body,
        grid=(_PGRID,),
        in_specs=[
            in_spec(0), in_spec(1), in_spec(2), in_spec(3),
            pl.BlockSpec((_PC, _D), lambda g: (0, 0)),
        ],
        out_specs=pl.BlockSpec((_VB, 4 * _PC), lambda g: (g, 0)),
        out_shape=jax.ShapeDtypeStruct((_P, 4 * _PC), jnp.float32),
    )(table_t, table_t, table_t, table_t, w32t)


def _gather_body(idx_hbm, ptab_hbm, out_hbm, idx_v, buf_v, gsem):
    cid = lax.axis_index("c")
    sid = lax.axis_index("s")
    wid = sid * 2 + cid
    base = wid * _ROWS_PER_W
    pltpu.sync_copy(idx_hbm.at[wid], idx_v)

    def fire(j, slot):
        pltpu.async_copy(ptab_hbm.at[idx_v.at[j]], buf_v.at[slot], gsem.at[slot])

    for s in range(_NBUF):
        fire(s, s)

    def outer(j0, carry):
        for s in range(_NBUF):
            j = j0 * _NBUF + s
            pltpu.make_async_copy(
                ptab_hbm.at[idx_v.at[j]], buf_v.at[s], gsem.at[s]
            ).wait()
            pltpu.sync_copy(buf_v.at[s], out_hbm.at[pl.ds(base + j * _CHUNK, _CHUNK)])

            @pl.when(j + _NBUF < _CHUNKS)
            def _():
                fire(j + _NBUF, s)

        return carry

    lax.fori_loop(0, _CHUNKS // _NBUF, outer, 0)


@functools.cache
def _sc_gather_fn():
    return pl.kernel(
        _gather_body,
        out_type=jax.ShapeDtypeStruct((_N, _PC), jnp.float32),
        mesh=plsc.VectorSubcoreMesh(core_axis_name="c", subcore_axis_name="s"),
        scratch_types=[
            pltpu.VMEM((_CHUNKS, _CHUNK), jnp.int32),
            pltpu.VMEM((_NBUF, _CHUNK, _PC), jnp.float32),
            pltpu.SemaphoreType.DMA((_NBUF,)),
        ],
        compiler_params=pltpu.CompilerParams(use_tc_tiling_on_sc=False),
    )


_BQ = 256


def _win_body(x_ref, b_ref, o0_ref, o1_ref, o2_ref, o3_ref):
    xb = x_ref[...]
    z2 = jnp.zeros((2, _BQ, 4 * _PC), jnp.float32)
    ppad = jnp.concatenate([z2, xb[: _L - 2], z2, xb[_L - 2 :]], axis=0)
    outs = (o0_ref, o1_ref, o2_ref, o3_ref)
    for j in range(4):
        acc = ppad[0:_L, :, 32 * j : 32 * j + _C]
        for i in range(1, 5):
            o = 32 * j + 6 * i
            acc = acc + ppad[i : i + _L, :, o : o + _C]
        acc = acc + b_ref[...]
        outs[j][...] = jnp.transpose(acc, (2, 0, 1))


def _window_sum(x4, bias):
    grid = _B // 4 // _BQ
    ospec = pl.BlockSpec((_C, _L, _BQ), lambda g: (0, 0, g))
    oshape = jax.ShapeDtypeStruct((_C, _L, _B // 4), jnp.float32)
    return pl.pallas_call(
        _win_body,
        grid=(grid,),
        in_specs=[
            pl.BlockSpec((_L, _BQ, 4 * _PC), lambda g: (0, g, 0)),
            pl.BlockSpec((1, 1, _C), lambda g: (0, 0, 0)),
        ],
        out_specs=[ospec, ospec, ospec, ospec],
        out_shape=[oshape, oshape, oshape, oshape],
    )(x4, bias)


def kernel(word, embed_table, W, b):
    table_t = jnp.transpose(embed_table)
    w_taps = W.reshape(_C, 5, _D)
    w32t = jnp.zeros((_PC, _D), jnp.float32)
    for i in range(5):
        w32t = w32t.at[6 * i : 6 * i + _C, :].set(w_taps[:, i, :])

    ptab = _project_table(table_t, w32t)
    ptab_lin = ptab.reshape(4 * _P, _PC)

    word_t = jnp.transpose(word)
    v = word_t.astype(jnp.int32).reshape(-1)
    idx = 4 * (v % _P) + v // _P
    idx = idx.reshape(_NW, _CHUNKS, _CHUNK)

    x = _sc_gather_fn()(idx, ptab_lin)
    x4 = x.reshape(_L, _B // 4, 4 * _PC)

    outs = _window_sum(x4, b.reshape(1, 1, _C))
    out_t = jnp.stack(outs, axis=-1).reshape(_C, _L, _B)
    return jnp.transpose(out_t, (2, 1, 0))


# speedup vs baseline: 2.7805x; 1.1498x over previous
import functools

import jax
import jax.numpy as jnp
from jax import lax
from jax.experimental import pallas as pl
from jax.experimental.pallas import tpu as pltpu
from jax.experimental.pallas import tpu_sc as plsc

_B, _L, _V, _D, _C = 4096, 50, 1000000, 64, 5
_N = _B * _L
_NW = 32
_ROWS_PER_W = _N // _NW
_CHUNK = 128
_CHUNKS = _ROWS_PER_W // _CHUNK
_NBUF = 5

_PC = 32
_VB = 2048
_PGRID = 123
_P = _PGRID * _VB
_TBLOCKS = (_V + _VB - 1) // _VB


def _proj_body(t0_ref, t1_ref, t2_ref, t3_ref, w_ref, out_ref):
    pieces = []
    for t_ref in (t0_ref, t1_ref, t2_ref, t3_ref):
        pieces.append(
            lax.dot_general(t_ref[...], w_ref[...], (((0,), (1,)), ((), ())),
                            preferred_element_type=jnp.float32)
        )
    out_ref[...] = jnp.concatenate(pieces, axis=1)


def _project_table(table_t, w32t):
    def in_spec(j):
        return pl.BlockSpec(
            (_D, _VB),
            lambda g, j=j: (0, jnp.minimum(g + j * _PGRID, _TBLOCKS - 1)),
        )

    return pl.pallas_call(
        _proj_body,
        grid=(_PGRID,),
        in_specs=[
            in_spec(0), in_spec(1), in_spec(2), in_spec(3),
            pl.BlockSpec((_PC, _D), lambda g: (0, 0)),
        ],
        out_specs=pl.BlockSpec((_VB, 4 * _PC), lambda g: (g, 0)),
        out_shape=jax.ShapeDtypeStruct((_P, 4 * _PC), jnp.float32),
    )(table_t, table_t, table_t, table_t, w32t)


def _gather_body(idx_hbm, ptab_hbm, out_hbm, idx_v, buf_v, gsem):
    cid = lax.axis_index("c")
    sid = lax.axis_index("s")
    wid = sid * 2 + cid
    base = wid * _ROWS_PER_W
    pltpu.sync_copy(idx_hbm.at[wid], idx_v)

    def fire(j, slot):
        pltpu.async_copy(ptab_hbm.at[idx_v.at[j]], buf_v.at[slot], gsem.at[slot])

    for s in range(_NBUF):
        fire(s, s)

    def outer(j0, carry):
        for s in range(_NBUF):
            j = j0 * _NBUF + s
            pltpu.make_async_copy(
                ptab_hbm.at[idx_v.at[j]], buf_v.at[s], gsem.at[s]
            ).wait()
            pltpu.sync_copy(buf_v.at[s], out_hbm.at[pl.ds(base + j * _CHUNK, _CHUNK)])

            @pl.when(j + _NBUF < _CHUNKS)
            def _():
                fire(j + _NBUF, s)

        return carry

    lax.fori_loop(0, _CHUNKS // _NBUF, outer, 0)


@functools.cache
def _sc_gather_fn():
    return pl.kernel(
        _gather_body,
        out_type=jax.ShapeDtypeStruct((_N, _PC), jnp.float32),
        mesh=plsc.VectorSubcoreMesh(core_axis_name="c", subcore_axis_name="s"),
        scratch_types=[
            pltpu.VMEM((_CHUNKS, _CHUNK), jnp.int32),
            pltpu.VMEM((_NBUF, _CHUNK, _PC), jnp.float32),
            pltpu.SemaphoreType.DMA((_NBUF,)),
        ],
        compiler_params=pltpu.CompilerParams(use_tc_tiling_on_sc=False),
    )


_BQ = 128


def _win_body(x_ref, b_ref, o0_ref, o1_ref, o2_ref, o3_ref):
    xb = x_ref[...]
    z2 = jnp.zeros((2, _BQ, 4 * _PC), jnp.float32)
    ppad = jnp.concatenate([z2, xb[: _L - 2], z2, xb[_L - 2 :]], axis=0)
    outs = (o0_ref, o1_ref, o2_ref, o3_ref)
    for j in range(4):
        acc = ppad[0:_L, :, 32 * j : 32 * j + _C]
        for i in range(1, 5):
            o = 32 * j + 6 * i
            acc = acc + ppad[i : i + _L, :, o : o + _C]
        acc = acc + b_ref[...]
        outs[j][...] = jnp.transpose(acc, (2, 0, 1))


def _window_sum(x4, bias):
    grid = _B // 4 // _BQ
    ospec = pl.BlockSpec((_C, _L, _BQ), lambda g: (0, 0, g))
    oshape = jax.ShapeDtypeStruct((_C, _L, _B // 4), jnp.float32)
    return pl.pallas_call(
        _win_body,
        grid=(grid,),
        in_specs=[
            pl.BlockSpec((_L, _BQ, 4 * _PC), lambda g: (0, g, 0)),
            pl.BlockSpec((1, 1, _C), lambda g: (0, 0, 0)),
        ],
        out_specs=[ospec, ospec, ospec, ospec],
        out_shape=[oshape, oshape, oshape, oshape],
    )(x4, bias)


def kernel(word, embed_table, W, b):
    table_t = jnp.transpose(embed_table)
    w_taps = W.reshape(_C, 5, _D)
    w32t = jnp.zeros((_PC, _D), jnp.float32)
    for i in range(5):
        w32t = w32t.at[6 * i : 6 * i + _C, :].set(w_taps[:, i, :])

    ptab = _project_table(table_t, w32t)
    ptab_lin = ptab.reshape(4 * _P, _PC)

    word_t = jnp.transpose(word)
    v = word_t.astype(jnp.int32).reshape(-1)
    idx = 4 * (v % _P) + v // _P
    idx = idx.reshape(_NW, _CHUNKS, _CHUNK)

    x = _sc_gather_fn()(idx, ptab_lin)
    x4 = x.reshape(_L, _B // 4, 4 * _PC)

    outs = _window_sum(x4, b.reshape(1, 1, _C))
    out_t = jnp.stack(outs, axis=-1).reshape(_C, _L, _B)
    return jnp.transpose(out_t, (2, 1, 0))


# speedup vs baseline: 3.0898x; 1.1112x over previous
import functools

import jax
import jax.numpy as jnp
from jax import lax
from jax.experimental import pallas as pl
from jax.experimental.pallas import tpu as pltpu
from jax.experimental.pallas import tpu_sc as plsc

_B, _L, _V, _D, _C = 4096, 50, 1000000, 64, 5
_N = _B * _L
_NW = 32
_ROWS_PER_W = _N // _NW
_CHUNK = 128
_CHUNKS = _ROWS_PER_W // _CHUNK
_NBUF = 5

_PC = 32
_VB = 2048
_PGRID = 123
_P = _PGRID * _VB
_TBLOCKS = (_V + _VB - 1) // _VB


def _proj_body(t0_ref, t1_ref, t2_ref, t3_ref, w_ref, out_ref):
    pieces = []
    for t_ref in (t0_ref, t1_ref, t2_ref, t3_ref):
        pieces.append(
            lax.dot_general(t_ref[...], w_ref[...], (((0,), (1,)), ((), ())),
                            preferred_element_type=jnp.float32)
        )
    out_ref[...] = jnp.concatenate(pieces, axis=1)


def _project_table(table_t, w32t):
    def in_spec(j):
        return pl.BlockSpec(
            (_D, _VB),
            lambda g, j=j: (0, jnp.minimum(g + j * _PGRID, _TBLOCKS - 1)),
        )

    return pl.pallas_call(
        _proj_body,
        grid=(_PGRID,),
        in_specs=[
            in_spec(0), in_spec(1), in_spec(2), in_spec(3),
            pl.BlockSpec((_PC, _D), lambda g: (0, 0)),
        ],
        out_specs=pl.BlockSpec((_VB, 4 * _PC), lambda g: (g, 0)),
        out_shape=jax.ShapeDtypeStruct((_P, 4 * _PC), jnp.float32),
    )(table_t, table_t, table_t, table_t, w32t)


def _gather_body(idx_hbm, ptab_hbm, out_hbm, idx_v, buf_v, gsem):
    cid = lax.axis_index("c")
    sid = lax.axis_index("s")
    wid = sid * 2 + cid
    base = wid * _ROWS_PER_W
    pltpu.sync_copy(idx_hbm.at[wid], idx_v)

    def fire(j, slot):
        pltpu.async_copy(ptab_hbm.at[idx_v.at[j]], buf_v.at[slot], gsem.at[slot])

    for s in range(_NBUF):
        fire(s, s)

    def outer(j0, carry):
        for s in range(_NBUF):
            j = j0 * _NBUF + s
            pltpu.make_async_copy(
                ptab_hbm.at[idx_v.at[j]], buf_v.at[s], gsem.at[s]
            ).wait()
            pltpu.sync_copy(buf_v.at[s], out_hbm.at[pl.ds(base + j * _CHUNK, _CHUNK)])

            @pl.when(j + _NBUF < _CHUNKS)
            def _():
                fire(j + _NBUF, s)

        return carry

    lax.fori_loop(0, _CHUNKS // _NBUF, outer, 0)


@functools.cache
def _sc_gather_fn():
    return pl.kernel(
        _gather_body,
        out_type=jax.ShapeDtypeStruct((_N, _PC), jnp.float32),
        mesh=plsc.VectorSubcoreMesh(core_axis_name="c", subcore_axis_name="s"),
        scratch_types=[
            pltpu.VMEM((_CHUNKS, _CHUNK), jnp.int32),
            pltpu.VMEM((_NBUF, _CHUNK, _PC), jnp.float32),
            pltpu.SemaphoreType.DMA((_NBUF,)),
        ],
        compiler_params=pltpu.CompilerParams(use_tc_tiling_on_sc=False),
    )


_BQ = 128


def _win_body(x_ref, b_ref, o0_ref, o1_ref, o2_ref, o3_ref):
    xb = x_ref[...]
    z2 = jnp.zeros((2, _BQ, 4 * _PC), jnp.float32)
    ppad = jnp.concatenate([z2, xb[: _L - 2], z2, xb[_L - 2 :]], axis=0)
    zl = jnp.zeros((_L, _BQ, 24), jnp.float32)
    s = ppad[0:_L]
    for i in range(1, 5):
        sh = jnp.concatenate([ppad[i : i + _L, :, 6 * i :], zl[:, :, : 6 * i]],
                             axis=2)
        s = s + sh
    st = jnp.transpose(s, (2, 0, 1))
    outs = (o0_ref, o1_ref, o2_ref, o3_ref)
    for j in range(4):
        outs[j][...] = st[32 * j : 32 * j + _C] + b_ref[...]


def _window_sum(x4, bias):
    grid = _B // 4 // _BQ
    ospec = pl.BlockSpec((_C, _L, _BQ), lambda g: (0, 0, g))
    oshape = jax.ShapeDtypeStruct((_C, _L, _B // 4), jnp.float32)
    return pl.pallas_call(
        _win_body,
        grid=(grid,),
        in_specs=[
            pl.BlockSpec((_L, _BQ, 4 * _PC), lambda g: (0, g, 0)),
            pl.BlockSpec((_C, 1, 1), lambda g: (0, 0, 0)),
        ],
        out_specs=[ospec, ospec, ospec, ospec],
        out_shape=[oshape, oshape, oshape, oshape],
    )(x4, bias)


def kernel(word, embed_table, W, b):
    table_t = jnp.transpose(embed_table)
    w_taps = W.reshape(_C, 5, _D)
    w32t = jnp.zeros((_PC, _D), jnp.float32)
    for i in range(5):
        w32t = w32t.at[6 * i : 6 * i + _C, :].set(w_taps[:, i, :])

    ptab = _project_table(table_t, w32t)
    ptab_lin = ptab.reshape(4 * _P, _PC)

    word_t = jnp.transpose(word)
    v = word_t.astype(jnp.int32).reshape(-1)
    idx = 4 * (v % _P) + v // _P
    idx = idx.reshape(_NW, _CHUNKS, _CHUNK)

    x = _sc_gather_fn()(idx, ptab_lin)
    x4 = x.reshape(_L, _B // 4, 4 * _PC)

    outs = _window_sum(x4, b.reshape(_C, 1, 1))
    out_t = jnp.stack(outs, axis=-1).reshape(_C, _L, _B)
    return jnp.transpose(out_t, (2, 1, 0))


# speedup vs baseline: 4.0861x; 1.3225x over previous
import functools

import jax
import jax.numpy as jnp
from jax import lax
from jax.experimental import pallas as pl
from jax.experimental.pallas import tpu as pltpu
from jax.experimental.pallas import tpu_sc as plsc

_B, _L, _V, _D, _C = 4096, 50, 1000000, 64, 5
_N = _B * _L
_NW = 32
_ROWS_PER_W = _N // _NW
_CHUNK = 128
_CHUNKS = _ROWS_PER_W // _CHUNK
_NBUF = 5

_PC = 32
_VB = 2048
_PGRID = 123
_P = _PGRID * _VB
_TBLOCKS = (_V + _VB - 1) // _VB


def _proj_body(t0_ref, t1_ref, t2_ref, t3_ref, w_ref, out_ref):
    pieces = []
    for t_ref in (t0_ref, t1_ref, t2_ref, t3_ref):
        pieces.append(
            jnp.dot(w_ref[...], t_ref[...], preferred_element_type=jnp.float32)
        )
    pjt = jnp.concatenate(pieces, axis=0)
    out_ref[...] = pjt.T


def _project_table(table_t, w32t):
    def in_spec(j):
        return pl.BlockSpec(
            (_D, _VB),
            lambda g, j=j: (0, jnp.minimum(g + j * _PGRID, _TBLOCKS - 1)),
        )

    return pl.pallas_call(
        _proj_body,
        grid=(_PGRID,),
        in_specs=[
            in_spec(0), in_spec(1), in_spec(2), in_spec(3),
            pl.BlockSpec((_PC, _D), lambda g: (0, 0)),
        ],
        out_specs=pl.BlockSpec((_VB, 4 * _PC), lambda g: (g, 0)),
        out_shape=jax.ShapeDtypeStruct((_P, 4 * _PC), jnp.float32),
    )(table_t, table_t, table_t, table_t, w32t)


def _gather_body(idx_hbm, ptab_hbm, out_hbm, idx_v, buf_v, gsem):
    cid = lax.axis_index("c")
    sid = lax.axis_index("s")
    wid = sid * 2 + cid
    base = wid * _ROWS_PER_W
    pltpu.sync_copy(idx_hbm.at[wid], idx_v)

    def fire(j, slot):
        pltpu.async_copy(ptab_hbm.at[idx_v.at[j]], buf_v.at[slot], gsem.at[slot])

    for s in range(_NBUF):
        fire(s, s)

    def outer(j0, carry):
        for s in range(_NBUF):
            j = j0 * _NBUF + s
            pltpu.make_async_copy(
                ptab_hbm.at[idx_v.at[j]], buf_v.at[s], gsem.at[s]
            ).wait()
            pltpu.sync_copy(buf_v.at[s], out_hbm.at[pl.ds(base + j * _CHUNK, _CHUNK)])

            @pl.when(j + _NBUF < _CHUNKS)
            def _():
                fire(j + _NBUF, s)

        return carry

    lax.fori_loop(0, _CHUNKS // _NBUF, outer, 0)


@functools.cache
def _sc_gather_fn():
    return pl.kernel(
        _gather_body,
        out_type=jax.ShapeDtypeStruct((_N, _PC), jnp.float32),
        mesh=plsc.VectorSubcoreMesh(core_axis_name="c", subcore_axis_name="s"),
        scratch_types=[
            pltpu.VMEM((_CHUNKS, _CHUNK), jnp.int32),
            pltpu.VMEM((_NBUF, _CHUNK, _PC), jnp.float32),
            pltpu.SemaphoreType.DMA((_NBUF,)),
        ],
        compiler_params=pltpu.CompilerParams(use_tc_tiling_on_sc=False),
    )


_BQ = 128


def _win_body(x_ref, b_ref, o0_ref, o1_ref, o2_ref, o3_ref):
    xb = x_ref[...]
    z2 = jnp.zeros((2, _BQ, 4 * _PC), jnp.float32)
    ppad = jnp.concatenate([z2, xb[: _L - 2], z2, xb[_L - 2 :]], axis=0)
    zl = jnp.zeros((_L, _BQ, 24), jnp.float32)
    s = ppad[0:_L]
    for i in range(1, 5):
        sh = jnp.concatenate([ppad[i : i + _L, :, 6 * i :], zl[:, :, : 6 * i]],
                             axis=2)
        s = s + sh
    st = jnp.transpose(s, (2, 0, 1))
    outs = (o0_ref, o1_ref, o2_ref, o3_ref)
    for j in range(4):
        outs[j][...] = st[32 * j : 32 * j + _C] + b_ref[...]


def _window_sum(x4, bias):
    grid = _B // 4 // _BQ
    ospec = pl.BlockSpec((_C, _L, _BQ), lambda g: (0, 0, g))
    oshape = jax.ShapeDtypeStruct((_C, _L, _B // 4), jnp.float32)
    return pl.pallas_call(
        _win_body,
        grid=(grid,),
        in_specs=[
            pl.BlockSpec((_L, _BQ, 4 * _PC), lambda g: (0, g, 0)),
            pl.BlockSpec((_C, 1, 1), lambda g: (0, 0, 0)),
        ],
        out_specs=[ospec, ospec, ospec, ospec],
        out_shape=[oshape, oshape, oshape, oshape],
    )(x4, bias)


def kernel(word, embed_table, W, b):
    table_t = jnp.transpose(embed_table)
    w_taps = W.reshape(_C, 5, _D)
    w32t = jnp.zeros((_PC, _D), jnp.float32)
    for i in range(5):
        w32t = w32t.at[6 * i : 6 * i + _C, :].set(w_taps[:, i, :])

    ptab = _project_table(table_t, w32t)
    ptab_lin = ptab.reshape(4 * _P, _PC)

    word_t = jnp.transpose(word)
    v = word_t.astype(jnp.int32).reshape(-1)
    idx = 4 * (v % _P) + v // _P
    idx = idx.reshape(_NW, _CHUNKS, _CHUNK)

    x = _sc_gather_fn()(idx, ptab_lin)
    x4 = x.reshape(_L, _B // 4, 4 * _PC)

    outs = _window_sum(x4, b.reshape(_C, 1, 1))
    out_t = jnp.stack(outs, axis=-1).reshape(_C, _L, _B)
    return jnp.transpose(out_t, (2, 1, 0))


# speedup vs baseline: 4.7513x; 1.1628x over previous
import functools

import jax
import jax.numpy as jnp
from jax import lax
from jax.experimental import pallas as pl
from jax.experimental.pallas import tpu as pltpu
from jax.experimental.pallas import tpu_sc as plsc

_B, _L, _V, _D, _C = 4096, 50, 1000000, 64, 5
_N = _B * _L
_NW = 32
_ROWS_PER_W = _N // _NW
_CHUNK = 128
_CHUNKS = _ROWS_PER_W // _CHUNK
_NBUF = 5

_PC = 32
_VB = 4096
_PGRID = 62
_P = _PGRID * _VB
_TBLOCKS = (_V + _VB - 1) // _VB


def _proj_body(t0_ref, t1_ref, t2_ref, t3_ref, w_ref, out_ref):
    pieces = []
    for t_ref in (t0_ref, t1_ref, t2_ref, t3_ref):
        pieces.append(
            jnp.dot(w_ref[...], t_ref[...], preferred_element_type=jnp.float32)
        )
    pjt = jnp.concatenate(pieces, axis=0)
    out_ref[...] = pjt.T


def _project_table(table_t, w32t):
    def in_spec(j):
        return pl.BlockSpec(
            (_D, _VB),
            lambda g, j=j: (0, jnp.minimum(g + j * _PGRID, _TBLOCKS - 1)),
        )

    return pl.pallas_call(
        _proj_body,
        grid=(_PGRID,),
        in_specs=[
            in_spec(0), in_spec(1), in_spec(2), in_spec(3),
            pl.BlockSpec((_PC, _D), lambda g: (0, 0)),
        ],
        out_specs=pl.BlockSpec((_VB, 4 * _PC), lambda g: (g, 0)),
        out_shape=jax.ShapeDtypeStruct((_P, 4 * _PC), jnp.float32),
    )(table_t, table_t, table_t, table_t, w32t)


def _gather_body(idx_hbm, ptab_hbm, out_hbm, idx_v, buf_v, gsem):
    cid = lax.axis_index("c")
    sid = lax.axis_index("s")
    wid = sid * 2 + cid
    base = wid * _ROWS_PER_W
    pltpu.sync_copy(idx_hbm.at[wid], idx_v)

    def fire(j, slot):
        pltpu.async_copy(ptab_hbm.at[idx_v.at[j]], buf_v.at[slot], gsem.at[slot])

    for s in range(_NBUF):
        fire(s, s)

    def outer(j0, carry):
        for s in range(_NBUF):
            j = j0 * _NBUF + s
            pltpu.make_async_copy(
                ptab_hbm.at[idx_v.at[j]], buf_v.at[s], gsem.at[s]
            ).wait()
            pltpu.sync_copy(buf_v.at[s], out_hbm.at[pl.ds(base + j * _CHUNK, _CHUNK)])

            @pl.when(j + _NBUF < _CHUNKS)
            def _():
                fire(j + _NBUF, s)

        return carry

    lax.fori_loop(0, _CHUNKS // _NBUF, outer, 0)


@functools.cache
def _sc_gather_fn():
    return pl.kernel(
        _gather_body,
        out_type=jax.ShapeDtypeStruct((_N, _PC), jnp.float32),
        mesh=plsc.VectorSubcoreMesh(core_axis_name="c", subcore_axis_name="s"),
        scratch_types=[
            pltpu.VMEM((_CHUNKS, _CHUNK), jnp.int32),
            pltpu.VMEM((_NBUF, _CHUNK, _PC), jnp.float32),
            pltpu.SemaphoreType.DMA((_NBUF,)),
        ],
        compiler_params=pltpu.CompilerParams(use_tc_tiling_on_sc=False),
    )


_BQ = 128


def _win_body(x_ref, b_ref, o0_ref, o1_ref, o2_ref, o3_ref):
    xb = x_ref[...]
    z2 = jnp.zeros((2, _BQ, 4 * _PC), jnp.float32)
    ppad = jnp.concatenate([z2, xb[: _L - 2], z2, xb[_L - 2 :]], axis=0)
    zl = jnp.zeros((_L, _BQ, 24), jnp.float32)
    s = ppad[0:_L]
    for i in range(1, 5):
        sh = jnp.concatenate([ppad[i : i + _L, :, 6 * i :], zl[:, :, : 6 * i]],
                             axis=2)
        s = s + sh
    st = jnp.transpose(s, (2, 0, 1))
    outs = (o0_ref, o1_ref, o2_ref, o3_ref)
    for j in range(4):
        outs[j][...] = st[32 * j : 32 * j + _C] + b_ref[...]


def _window_sum(x4, bias):
    grid = _B // 4 // _BQ
    ospec = pl.BlockSpec((_C, _L, _BQ), lambda g: (0, 0, g))
    oshape = jax.ShapeDtypeStruct((_C, _L, _B // 4), jnp.float32)
    return pl.pallas_call(
        _win_body,
        grid=(grid,),
        in_specs=[
            pl.BlockSpec((_L, _BQ, 4 * _PC), lambda g: (0, g, 0)),
            pl.BlockSpec((_C, 1, 1), lambda g: (0, 0, 0)),
        ],
        out_specs=[ospec, ospec, ospec, ospec],
        out_shape=[oshape, oshape, oshape, oshape],
    )(x4, bias)


def kernel(word, embed_table, W, b):
    table_t = jnp.transpose(embed_table)
    w_taps = W.reshape(_C, 5, _D)
    w32t = jnp.zeros((_PC, _D), jnp.float32)
    for i in range(5):
        w32t = w32t.at[6 * i : 6 * i + _C, :].set(w_taps[:, i, :])

    ptab = _project_table(table_t, w32t)
    ptab_lin = ptab.reshape(4 * _P, _PC)

    word_t = jnp.transpose(word)
    v = word_t.astype(jnp.int32).reshape(-1)
    idx = 4 * (v % _P) + v // _P
    idx = idx.reshape(_NW, _CHUNKS, _CHUNK)

    x = _sc_gather_fn()(idx, ptab_lin)
    x4 = x.reshape(_L, _B // 4, 4 * _PC)

    outs = _window_sum(x4, b.reshape(_C, 1, 1))
    out_t = jnp.stack(outs, axis=-1).reshape(_C, _L, _B)
    return jnp.transpose(out_t, (2, 1, 0))


# speedup vs baseline: 4.9032x; 1.0320x over previous
import functools

import jax
import jax.numpy as jnp
from jax import lax
from jax.experimental import pallas as pl
from jax.experimental.pallas import tpu as pltpu
from jax.experimental.pallas import tpu_sc as plsc

_B, _L, _V, _D, _C = 4096, 50, 1000000, 64, 5
_N = _B * _L
_NW = 32
_ROWS_PER_W = _N // _NW
_CHUNK = 128
_CHUNKS = _ROWS_PER_W // _CHUNK
_NBUF = 5

_PC = 32
_VB = 8192
_PGRID = 31
_P = _PGRID * _VB
_TBLOCKS = (_V + _VB - 1) // _VB


def _proj_body(t0_ref, t1_ref, t2_ref, t3_ref, w_ref, out_ref):
    pieces = []
    for t_ref in (t0_ref, t1_ref, t2_ref, t3_ref):
        pieces.append(
            jnp.dot(w_ref[...], t_ref[...], preferred_element_type=jnp.float32)
        )
    pjt = jnp.concatenate(pieces, axis=0)
    out_ref[...] = pjt.T


def _project_table(table_t, w32t):
    def in_spec(j):
        return pl.BlockSpec(
            (_D, _VB),
            lambda g, j=j: (0, jnp.minimum(g + j * _PGRID, _TBLOCKS - 1)),
        )

    return pl.pallas_call(
        _proj_body,
        grid=(_PGRID,),
        in_specs=[
            in_spec(0), in_spec(1), in_spec(2), in_spec(3),
            pl.BlockSpec((_PC, _D), lambda g: (0, 0)),
        ],
        out_specs=pl.BlockSpec((_VB, 4 * _PC), lambda g: (g, 0)),
        out_shape=jax.ShapeDtypeStruct((_P, 4 * _PC), jnp.float32),
    )(table_t, table_t, table_t, table_t, w32t)


def _gather_body(idx_hbm, ptab_hbm, out_hbm, idx_v, buf_v, gsem):
    cid = lax.axis_index("c")
    sid = lax.axis_index("s")
    wid = sid * 2 + cid
    base = wid * _ROWS_PER_W
    pltpu.sync_copy(idx_hbm.at[wid], idx_v)

    def fire(j, slot):
        pltpu.async_copy(ptab_hbm.at[idx_v.at[j]], buf_v.at[slot], gsem.at[slot])

    for s in range(_NBUF):
        fire(s, s)

    def outer(j0, carry):
        for s in range(_NBUF):
            j = j0 * _NBUF + s
            pltpu.make_async_copy(
                ptab_hbm.at[idx_v.at[j]], buf_v.at[s], gsem.at[s]
            ).wait()
            pltpu.sync_copy(buf_v.at[s], out_hbm.at[pl.ds(base + j * _CHUNK, _CHUNK)])

            @pl.when(j + _NBUF < _CHUNKS)
            def _():
                fire(j + _NBUF, s)

        return carry

    lax.fori_loop(0, _CHUNKS // _NBUF, outer, 0)


@functools.cache
def _sc_gather_fn():
    return pl.kernel(
        _gather_body,
        out_type=jax.ShapeDtypeStruct((_N, _PC), jnp.float32),
        mesh=plsc.VectorSubcoreMesh(core_axis_name="c", subcore_axis_name="s"),
        scratch_types=[
            pltpu.VMEM((_CHUNKS, _CHUNK), jnp.int32),
            pltpu.VMEM((_NBUF, _CHUNK, _PC), jnp.float32),
            pltpu.SemaphoreType.DMA((_NBUF,)),
        ],
        compiler_params=pltpu.CompilerParams(use_tc_tiling_on_sc=False),
    )


_BQ = 256


def _win_body(x_ref, b_ref, o0_ref, o1_ref, o2_ref, o3_ref):
    xb = x_ref[...]
    z2 = jnp.zeros((2, _BQ, 4 * _PC), jnp.float32)
    ppad = jnp.concatenate([z2, xb[: _L - 2], z2, xb[_L - 2 :]], axis=0)
    zl = jnp.zeros((_L, _BQ, 24), jnp.float32)
    s = ppad[0:_L]
    for i in range(1, 5):
        sh = jnp.concatenate([ppad[i : i + _L, :, 6 * i :], zl[:, :, : 6 * i]],
                             axis=2)
        s = s + sh
    st = jnp.transpose(s, (2, 0, 1))
    outs = (o0_ref, o1_ref, o2_ref, o3_ref)
    for j in range(4):
        outs[j][...] = st[32 * j : 32 * j + _C] + b_ref[...]


def _window_sum(x4, bias):
    grid = _B // 4 // _BQ
    ospec = pl.BlockSpec((_C, _L, _BQ), lambda g: (0, 0, g))
    oshape = jax.ShapeDtypeStruct((_C, _L, _B // 4), jnp.float32)
    return pl.pallas_call(
        _win_body,
        grid=(grid,),
        in_specs=[
            pl.BlockSpec((_L, _BQ, 4 * _PC), lambda g: (0, g, 0)),
            pl.BlockSpec((_C, 1, 1), lambda g: (0, 0, 0)),
        ],
        out_specs=[ospec, ospec, ospec, ospec],
        out_shape=[oshape, oshape, oshape, oshape],
    )(x4, bias)


def kernel(word, embed_table, W, b):
    table_t = jnp.transpose(embed_table)
    w_taps = W.reshape(_C, 5, _D)
    w32t = jnp.zeros((_PC, _D), jnp.float32)
    for i in range(5):
        w32t = w32t.at[6 * i : 6 * i + _C, :].set(w_taps[:, i, :])

    ptab = _project_table(table_t, w32t)
    ptab_lin = ptab.reshape(4 * _P, _PC)

    word_t = jnp.transpose(word)
    v = word_t.astype(jnp.int32).reshape(-1)
    idx = 4 * (v % _P) + v // _P
    idx = idx.reshape(_NW, _CHUNKS, _CHUNK)

    x = _sc_gather_fn()(idx, ptab_lin)
    x4 = x.reshape(_L, _B // 4, 4 * _PC)

    outs = _window_sum(x4, b.reshape(_C, 1, 1))
    out_t = jnp.stack(outs, axis=-1).reshape(_C, _L, _B)
    return jnp.transpose(out_t, (2, 1, 0))
